# baseline (speedup 1.0000x reference)
import jax
import jax.numpy as jnp
from jax import lax
from jax.experimental import pallas as pl
from jax.experimental.pallas import tpu as pltpu
from jax.experimental.pallas import tpu_sc as plsc

_N = 320000
_D = 128
_S = 512
_NC = 2
_NS = 16
_L = 16
_NW = _NC * _NS

_NSC = 153600
_RPW = _NSC // _NW
_T = 400
_NCHUNK = _RPW // _T
_G = _T // _L
_SPAD = 513
_TAU = (0, 8, 4, 12, 2, 10, 6, 14, 1, 9, 5, 13, 3, 11, 7, 15)

_RTC = 2560
_NTC = _N - _NSC
_NBLK = _NTC // _RTC
_YPW = _NTC // _NW
_YG = _YPW // _L


def _sc_partials(x_flat, ids, w_vec):
    mesh = plsc.VectorSubcoreMesh(
        core_axis_name="c", subcore_axis_name="s",
        num_cores=_NC, num_subcores=_NS)

    def body(x_hbm, ids_hbm, w_hbm, out_hbm,
             xb0, xb1, ids_v, w_v, acc2, acc_v, sem0, sem1):
        cid = lax.axis_index("c")
        sid = lax.axis_index("s")
        wid = sid * _NC + cid
        base_row = wid * _RPW

        def dcopy(c, buf_ref, sem):
            return pltpu.make_async_copy(
                x_hbm.at[pl.ds((base_row + c * _T) * _D, _T * _D)],
                buf_ref, sem)

        dcopy(0, xb0, sem0).start()
        pltpu.sync_copy(ids_hbm.at[pl.ds(base_row, _RPW)], ids_v)
        pltpu.sync_copy(w_hbm, w_v)
        w_regs = [w_v[pl.ds(k * _L, _L)] for k in range(_D // _L)]

        zero = jnp.zeros((_L,), jnp.float32)
        lanes = jnp.arange(_L, dtype=jnp.int32)
        lane_base = lanes * _SPAD
        masks = {h: (lanes % (2 * h)) < h for h in (8, 4, 2, 1)}
        rot_idx = {
            h: ((lanes + h) & (_L - 1), (lanes - h) & (_L - 1))
            for h in (8, 4, 2, 1)
        }

        def take(v, idx):
            return v.at[idx].get(mode="promise_in_bounds", unique_indices=True)

        def zero_body(i, carry):
            acc2[pl.ds(i * _L, _L)] = zero
            return carry

        lax.fori_loop(0, (_NS * _SPAD) // _L, zero_body, 0)

        def compute(xb, c):
            def group_body(g, carry):
                idv = ids_v[pl.ds(c * _T + g * _L, _L)]
                vs = []
                for j in range(_L):
                    base = (g * _L + _TAU[j]) * _D
                    p = xb[pl.ds(base, _L)] * w_regs[0]
                    for k in range(1, _D // _L):
                        p = p + xb[pl.ds(base + k * _L, _L)] * w_regs[k]
                    vs.append(p)
                for h in (8, 4, 2, 1):
                    m = masks[h]
                    ip, im = rot_idx[h]
                    vs = [jnp.where(m, vs[i2], take(vs[i2 + 1], im))
                          + jnp.where(m, take(vs[i2], ip), vs[i2 + 1])
                          for i2 in range(0, len(vs), 2)]
                plsc.addupdate_scatter(acc2, [lane_base + idv], vs[0])
                return carry

            lax.fori_loop(0, _G, group_body, 0)

        def ring_body(i, carry):
            c0 = 2 * i
            dcopy(c0 + 1, xb1, sem1).start()
            dcopy(c0, xb0, sem0).wait()
            compute(xb0, c0)
            dcopy(c0 + 2, xb0, sem0).start()
            dcopy(c0 + 1, xb1, sem1).wait()
            compute(xb1, c0 + 1)
            return carry

        lax.fori_loop(0, (_NCHUNK - 1) // 2, ring_body, 0)
        if _NCHUNK % 2 == 1:
            dcopy(_NCHUNK - 1, xb0, sem0).wait()
            compute(xb0, _NCHUNK - 1)
        else:
            dcopy(_NCHUNK - 1, xb1, sem1).start()
            dcopy(_NCHUNK - 2, xb0, sem0).wait()
            compute(xb0, _NCHUNK - 2)
            dcopy(_NCHUNK - 1, xb1, sem1).wait()
            compute(xb1, _NCHUNK - 1)

        def fold_body(cg, carry):
            s = acc2[pl.ds(cg * _L, _L)]
            for r in range(1, _NS):
                s = s + acc2[pl.ds(r * _SPAD + cg * _L, _L)]
            acc_v[pl.ds(cg * _L, _L)] = s
            return carry

        lax.fori_loop(0, _S // _L, fold_body, 0)
        pltpu.sync_copy(acc_v, out_hbm.at[pl.ds(wid * _S, _S)])

    f = pl.kernel(
        body,
        out_type=jax.ShapeDtypeStruct((_NW * _S,), jnp.float32),
        mesh=mesh,
        compiler_params=pltpu.CompilerParams(needs_layout_passes=False),
        scratch_types=[
            pltpu.VMEM((_T * _D,), jnp.float32),
            pltpu.VMEM((_T * _D,), jnp.float32),
            pltpu.VMEM((_RPW,), jnp.int32),
            pltpu.VMEM((_D,), jnp.float32),
            pltpu.VMEM((_NS * _SPAD,), jnp.float32),
            pltpu.VMEM((_S,), jnp.float32),
            pltpu.SemaphoreType.DMA,
            pltpu.SemaphoreType.DMA,
        ],
    )
    return f(x_flat, ids, w_vec)


def _sc_scatter_y(y_flat, ids, sc_partials_flat):
    mesh = plsc.VectorSubcoreMesh(
        core_axis_name="c", subcore_axis_name="s",
        num_cores=_NC, num_subcores=_NS)

    def body(y_hbm, ids_hbm, part_hbm, out_hbm, y_v, ids_v, part_v, acc2, acc_v):
        cid = lax.axis_index("c")
        sid = lax.axis_index("s")
        wid = sid * _NC + cid
        base = wid * _YPW
        pltpu.sync_copy(y_hbm.at[pl.ds(base, _YPW)], y_v)
        pltpu.sync_copy(ids_hbm.at[pl.ds(_NSC + base, _YPW)], ids_v)
        pltpu.sync_copy(part_hbm.at[pl.ds(wid * _S, _S)], part_v)

        zero = jnp.zeros((_L,), jnp.float32)
        lanes = jnp.arange(_L, dtype=jnp.int32)
        lane_base = lanes * _SPAD

        def zero_body(i, carry):
            acc2[pl.ds(i * _L, _L)] = zero
            return carry

        lax.fori_loop(0, (_NS * _SPAD) // _L, zero_body, 0)

        def group_body(g, carry):
            yv = y_v[pl.ds(g * _L, _L)]
            idv = ids_v[pl.ds(g * _L, _L)]
            plsc.addupdate_scatter(acc2, [lane_base + idv], yv)
            return carry

        lax.fori_loop(0, _YG, group_body, 0)

        def fold_body(cg, carry):
            s = part_v[pl.ds(cg * _L, _L)]
            for r in range(_NS):
                s = s + acc2[pl.ds(r * _SPAD + cg * _L, _L)]
            acc_v[pl.ds(cg * _L, _L)] = s
            return carry

        lax.fori_loop(0, _S // _L, fold_body, 0)
        pltpu.sync_copy(acc_v, out_hbm.at[pl.ds(wid * _S, _S)])

    f = pl.kernel(
        body,
        out_type=jax.ShapeDtypeStruct((_NW * _S,), jnp.float32),
        mesh=mesh,
        compiler_params=pltpu.CompilerParams(needs_layout_passes=False),
        scratch_types=[
            pltpu.VMEM((_YPW,), jnp.float32),
            pltpu.VMEM((_YPW,), jnp.int32),
            pltpu.VMEM((_S,), jnp.float32),
            pltpu.VMEM((_NS * _SPAD,), jnp.float32),
            pltpu.VMEM((_S,), jnp.float32),
        ],
    )
    return f(y_flat, ids, sc_partials_flat)


def _tc_matvec(x_full, w_row):
    def body(x_ref, w_ref, o_ref):
        y = lax.dot_general(
            w_ref[...], x_ref[...],
            dimension_numbers=(((1,), (1,)), ((), ())),
            preferred_element_type=jnp.float32)
        o_ref[...] = y.reshape(1, 1, _RTC)

    blk0 = _NSC // _RTC

    return pl.pallas_call(
        body,
        grid=(_NBLK,),
        in_specs=[
            pl.BlockSpec((_RTC, _D), lambda i: (i + blk0, 0)),
            pl.BlockSpec((1, _D), lambda i: (0, 0)),
        ],
        out_specs=pl.BlockSpec((1, 1, _RTC), lambda i: (i, 0, 0)),
        out_shape=jax.ShapeDtypeStruct((_NBLK, 1, _RTC), jnp.float32),
        compiler_params=pltpu.CompilerParams(
            dimension_semantics=("parallel",)),
    )(x_full, w_row)


def _combine(partials, b2):
    def body(p_ref, b_ref, o_ref):
        o_ref[...] = jnp.sum(p_ref[...], axis=0, keepdims=True) + b_ref[0, 0]

    return pl.pallas_call(
        body,
        out_shape=jax.ShapeDtypeStruct((1, _S), jnp.float32),
    )(partials, b2)


def kernel(x, batch, W, b):
    x_flat = x.reshape(-1)
    w_vec = W.reshape(-1)
    ids = batch.astype(jnp.int32)
    sc_partials = _sc_partials(x_flat, ids, w_vec)
    y_tc = _tc_matvec(x, W.reshape(1, _D)).reshape(-1)
    partials = _sc_scatter_y(y_tc, ids, sc_partials).reshape(_NW, _S)
    out2 = _combine(partials, b.reshape(1, 1))
    return out2.reshape(_S)


# speedup vs baseline: 1.0988x; 1.0988x over previous
import jax
import jax.numpy as jnp
from jax import lax
from jax.experimental import pallas as pl
from jax.experimental.pallas import tpu as pltpu
from jax.experimental.pallas import tpu_sc as plsc

_N = 320000
_D = 128
_S = 512
_NC = 2
_NS = 16
_L = 16
_NW = _NC * _NS

_NSC = 192000
_RPW = _NSC // _NW
_T = 400
_NCHUNK = _RPW // _T
_G = _T // _L
_SPAD = 513
_TAU = (0, 8, 4, 12, 2, 10, 6, 14, 1, 9, 5, 13, 3, 11, 7, 15)

_RTC = 4000
_NTC = _N - _NSC
_NBLK = _NTC // _RTC
_YPW = _NTC // _NW
_YG = _YPW // _L


def _sc_partials(x_flat, ids, w_vec):
    mesh = plsc.VectorSubcoreMesh(
        core_axis_name="c", subcore_axis_name="s",
        num_cores=_NC, num_subcores=_NS)

    def body(x_hbm, ids_hbm, w_hbm, out_hbm,
             xb0, xb1, ids_v, w_v, acc2, acc_v, sem0, sem1):
        cid = lax.axis_index("c")
        sid = lax.axis_index("s")
        wid = sid * _NC + cid
        base_row = wid * _RPW

        def dcopy(c, buf_ref, sem):
            return pltpu.make_async_copy(
                x_hbm.at[pl.ds((base_row + c * _T) * _D, _T * _D)],
                buf_ref, sem)

        dcopy(0, xb0, sem0).start()
        pltpu.sync_copy(ids_hbm.at[pl.ds(base_row, _RPW)], ids_v)
        pltpu.sync_copy(w_hbm, w_v)
        w_regs = [w_v[pl.ds(k * _L, _L)] for k in range(_D // _L)]

        zero = jnp.zeros((_L,), jnp.float32)
        lanes = jnp.arange(_L, dtype=jnp.int32)
        lane_base = lanes * _SPAD
        masks = {h: (lanes % (2 * h)) < h for h in (8, 4, 2, 1)}
        rot_idx = {
            h: ((lanes + h) & (_L - 1), (lanes - h) & (_L - 1))
            for h in (8, 4, 2, 1)
        }

        def take(v, idx):
            return v.at[idx].get(mode="promise_in_bounds", unique_indices=True)

        def zero_body(i, carry):
            acc2[pl.ds(i * _L, _L)] = zero
            return carry

        lax.fori_loop(0, (_NS * _SPAD) // _L, zero_body, 0)

        def compute(xb, c):
            def group_body(g, carry):
                idv = ids_v[pl.ds(c * _T + g * _L, _L)]
                vs = []
                for j in range(_L):
                    base = (g * _L + _TAU[j]) * _D
                    p = xb[pl.ds(base, _L)] * w_regs[0]
                    for k in range(1, _D // _L):
                        p = p + xb[pl.ds(base + k * _L, _L)] * w_regs[k]
                    vs.append(p)
                for h in (8, 4, 2, 1):
                    m = masks[h]
                    ip, im = rot_idx[h]
                    vs = [jnp.where(m, vs[i2], take(vs[i2 + 1], im))
                          + jnp.where(m, take(vs[i2], ip), vs[i2 + 1])
                          for i2 in range(0, len(vs), 2)]
                plsc.addupdate_scatter(acc2, [lane_base + idv], vs[0])
                return carry

            lax.fori_loop(0, _G, group_body, 0)

        def ring_body(i, carry):
            c0 = 2 * i
            dcopy(c0 + 1, xb1, sem1).start()
            dcopy(c0, xb0, sem0).wait()
            compute(xb0, c0)
            dcopy(c0 + 2, xb0, sem0).start()
            dcopy(c0 + 1, xb1, sem1).wait()
            compute(xb1, c0 + 1)
            return carry

        lax.fori_loop(0, (_NCHUNK - 1) // 2, ring_body, 0)
        if _NCHUNK % 2 == 1:
            dcopy(_NCHUNK - 1, xb0, sem0).wait()
            compute(xb0, _NCHUNK - 1)
        else:
            dcopy(_NCHUNK - 1, xb1, sem1).start()
            dcopy(_NCHUNK - 2, xb0, sem0).wait()
            compute(xb0, _NCHUNK - 2)
            dcopy(_NCHUNK - 1, xb1, sem1).wait()
            compute(xb1, _NCHUNK - 1)

        def fold_body(cg, carry):
            s = acc2[pl.ds(cg * _L, _L)]
            for r in range(1, _NS):
                s = s + acc2[pl.ds(r * _SPAD + cg * _L, _L)]
            acc_v[pl.ds(cg * _L, _L)] = s
            return carry

        lax.fori_loop(0, _S // _L, fold_body, 0)
        pltpu.sync_copy(acc_v, out_hbm.at[pl.ds(wid * _S, _S)])

    f = pl.kernel(
        body,
        out_type=jax.ShapeDtypeStruct((_NW * _S,), jnp.float32),
        mesh=mesh,
        compiler_params=pltpu.CompilerParams(needs_layout_passes=False),
        scratch_types=[
            pltpu.VMEM((_T * _D,), jnp.float32),
            pltpu.VMEM((_T * _D,), jnp.float32),
            pltpu.VMEM((_RPW,), jnp.int32),
            pltpu.VMEM((_D,), jnp.float32),
            pltpu.VMEM((_NS * _SPAD,), jnp.float32),
            pltpu.VMEM((_S,), jnp.float32),
            pltpu.SemaphoreType.DMA,
            pltpu.SemaphoreType.DMA,
        ],
    )
    return f(x_flat, ids, w_vec)


def _sc_scatter_y(y_flat, ids, sc_partials_flat):
    mesh = plsc.VectorSubcoreMesh(
        core_axis_name="c", subcore_axis_name="s",
        num_cores=_NC, num_subcores=_NS)

    def body(y_hbm, ids_hbm, part_hbm, out_hbm, y_v, ids_v, part_v, acc2, acc_v):
        cid = lax.axis_index("c")
        sid = lax.axis_index("s")
        wid = sid * _NC + cid
        base = wid * _YPW
        pltpu.sync_copy(y_hbm.at[pl.ds(base, _YPW)], y_v)
        pltpu.sync_copy(ids_hbm.at[pl.ds(_NSC + base, _YPW)], ids_v)
        pltpu.sync_copy(part_hbm.at[pl.ds(wid * _S, _S)], part_v)

        zero = jnp.zeros((_L,), jnp.float32)
        lanes = jnp.arange(_L, dtype=jnp.int32)
        lane_base = lanes * _SPAD

        def zero_body(i, carry):
            acc2[pl.ds(i * _L, _L)] = zero
            return carry

        lax.fori_loop(0, (_NS * _SPAD) // _L, zero_body, 0)

        def group_body(g, carry):
            yv = y_v[pl.ds(g * _L, _L)]
            idv = ids_v[pl.ds(g * _L, _L)]
            plsc.addupdate_scatter(acc2, [lane_base + idv], yv)
            return carry

        lax.fori_loop(0, _YG, group_body, 0)

        def fold_body(cg, carry):
            s = part_v[pl.ds(cg * _L, _L)]
            for r in range(_NS):
                s = s + acc2[pl.ds(r * _SPAD + cg * _L, _L)]
            acc_v[pl.ds(cg * _L, _L)] = s
            return carry

        lax.fori_loop(0, _S // _L, fold_body, 0)
        pltpu.sync_copy(acc_v, out_hbm.at[pl.ds(wid * _S, _S)])

    f = pl.kernel(
        body,
        out_type=jax.ShapeDtypeStruct((_NW * _S,), jnp.float32),
        mesh=mesh,
        compiler_params=pltpu.CompilerParams(needs_layout_passes=False),
        scratch_types=[
            pltpu.VMEM((_YPW,), jnp.float32),
            pltpu.VMEM((_YPW,), jnp.int32),
            pltpu.VMEM((_S,), jnp.float32),
            pltpu.VMEM((_NS * _SPAD,), jnp.float32),
            pltpu.VMEM((_S,), jnp.float32),
        ],
    )
    return f(y_flat, ids, sc_partials_flat)


def _tc_matvec(x_full, w_row):
    def body(x_ref, w_ref, o_ref):
        y = lax.dot_general(
            w_ref[...], x_ref[...],
            dimension_numbers=(((1,), (1,)), ((), ())),
            preferred_element_type=jnp.float32)
        o_ref[...] = y.reshape(1, 1, _RTC)

    blk0 = _NSC // _RTC

    return pl.pallas_call(
        body,
        grid=(_NBLK,),
        in_specs=[
            pl.BlockSpec((_RTC, _D), lambda i: (i + blk0, 0)),
            pl.BlockSpec((1, _D), lambda i: (0, 0)),
        ],
        out_specs=pl.BlockSpec((1, 1, _RTC), lambda i: (i, 0, 0)),
        out_shape=jax.ShapeDtypeStruct((_NBLK, 1, _RTC), jnp.float32),
        compiler_params=pltpu.CompilerParams(
            dimension_semantics=("parallel",)),
    )(x_full, w_row)


def _combine(partials, b2):
    def body(p_ref, b_ref, o_ref):
        o_ref[...] = jnp.sum(p_ref[...], axis=0, keepdims=True) + b_ref[0, 0]

    return pl.pallas_call(
        body,
        out_shape=jax.ShapeDtypeStruct((1, _S), jnp.float32),
    )(partials, b2)


def kernel(x, batch, W, b):
    x_flat = x.reshape(-1)
    w_vec = W.reshape(-1)
    ids = batch.astype(jnp.int32)
    sc_partials = _sc_partials(x_flat, ids, w_vec)
    y_tc = _tc_matvec(x, W.reshape(1, _D)).reshape(-1)
    partials = _sc_scatter_y(y_tc, ids, sc_partials).reshape(_NW, _S)
    out2 = _combine(partials, b.reshape(1, 1))
    return out2.reshape(_S)


# speedup vs baseline: 1.1744x; 1.0688x over previous
import jax
import jax.numpy as jnp
from jax import lax
from jax.experimental import pallas as pl
from jax.experimental.pallas import tpu as pltpu
from jax.experimental.pallas import tpu_sc as plsc

_N = 320000
_D = 128
_S = 512
_NC = 2
_NS = 16
_L = 16
_NW = _NC * _NS

_NSC = 199680
_RPW = _NSC // _NW
_T = 416
_NCHUNK = _RPW // _T
_G = _T // _L
_SPAD = 513
_TAU = (0, 8, 4, 12, 2, 10, 6, 14, 1, 9, 5, 13, 3, 11, 7, 15)

_RTC = 2560
_NTC = _N - _NSC
_NBLK = _NTC // _RTC
_YPW = _NTC // _NW
_YG = _YPW // _L


def _sc_partials(x_flat, ids, w_vec):
    mesh = plsc.VectorSubcoreMesh(
        core_axis_name="c", subcore_axis_name="s",
        num_cores=_NC, num_subcores=_NS)

    def body(x_hbm, ids_hbm, w_hbm, out_hbm,
             xb0, xb1, ids_v, w_v, acc2, acc_v, sem0, sem1):
        cid = lax.axis_index("c")
        sid = lax.axis_index("s")
        wid = sid * _NC + cid
        base_row = wid * _RPW

        def dcopy(c, buf_ref, sem):
            return pltpu.make_async_copy(
                x_hbm.at[pl.ds((base_row + c * _T) * _D, _T * _D)],
                buf_ref, sem)

        dcopy(0, xb0, sem0).start()
        pltpu.sync_copy(ids_hbm.at[pl.ds(base_row, _RPW)], ids_v)
        pltpu.sync_copy(w_hbm, w_v)
        w_regs = [w_v[pl.ds(k * _L, _L)] for k in range(_D // _L)]

        zero = jnp.zeros((_L,), jnp.float32)
        lanes = jnp.arange(_L, dtype=jnp.int32)
        lane_base = lanes * _SPAD
        masks = {h: (lanes % (2 * h)) < h for h in (8, 4, 2, 1)}
        rot_idx = {
            h: ((lanes + h) & (_L - 1), (lanes - h) & (_L - 1))
            for h in (8, 4, 2, 1)
        }

        def take(v, idx):
            return v.at[idx].get(mode="promise_in_bounds", unique_indices=True)

        def zero_body(i, carry):
            acc2[pl.ds(i * _L, _L)] = zero
            return carry

        lax.fori_loop(0, (_NS * _SPAD) // _L, zero_body, 0)

        def compute(xb, c):
            def group_body(g, carry):
                idv = ids_v[pl.ds(c * _T + g * _L, _L)]
                vs = []
                for j in range(_L):
                    base = (g * _L + _TAU[j]) * _D
                    p = xb[pl.ds(base, _L)] * w_regs[0]
                    for k in range(1, _D // _L):
                        p = p + xb[pl.ds(base + k * _L, _L)] * w_regs[k]
                    vs.append(p)
                for h in (8, 4, 2, 1):
                    m = masks[h]
                    ip, im = rot_idx[h]
                    vs = [jnp.where(m, vs[i2], take(vs[i2 + 1], im))
                          + jnp.where(m, take(vs[i2], ip), vs[i2 + 1])
                          for i2 in range(0, len(vs), 2)]
                plsc.addupdate_scatter(acc2, [lane_base + idv], vs[0])
                return carry

            lax.fori_loop(0, _G, group_body, 0)

        def ring_body(i, carry):
            c0 = 2 * i
            dcopy(c0 + 1, xb1, sem1).start()
            dcopy(c0, xb0, sem0).wait()
            compute(xb0, c0)
            dcopy(c0 + 2, xb0, sem0).start()
            dcopy(c0 + 1, xb1, sem1).wait()
            compute(xb1, c0 + 1)
            return carry

        lax.fori_loop(0, (_NCHUNK - 1) // 2, ring_body, 0)
        if _NCHUNK % 2 == 1:
            dcopy(_NCHUNK - 1, xb0, sem0).wait()
            compute(xb0, _NCHUNK - 1)
        else:
            dcopy(_NCHUNK - 1, xb1, sem1).start()
            dcopy(_NCHUNK - 2, xb0, sem0).wait()
            compute(xb0, _NCHUNK - 2)
            dcopy(_NCHUNK - 1, xb1, sem1).wait()
            compute(xb1, _NCHUNK - 1)

        def fold_body(cg, carry):
            s = acc2[pl.ds(cg * _L, _L)]
            for r in range(1, _NS):
                s = s + acc2[pl.ds(r * _SPAD + cg * _L, _L)]
            acc_v[pl.ds(cg * _L, _L)] = s
            return carry

        lax.fori_loop(0, _S // _L, fold_body, 0)
        pltpu.sync_copy(acc_v, out_hbm.at[pl.ds(wid * _S, _S)])

    f = pl.kernel(
        body,
        out_type=jax.ShapeDtypeStruct((_NW * _S,), jnp.float32),
        mesh=mesh,
        compiler_params=pltpu.CompilerParams(needs_layout_passes=False),
        scratch_types=[
            pltpu.VMEM((_T * _D,), jnp.float32),
            pltpu.VMEM((_T * _D,), jnp.float32),
            pltpu.VMEM((_RPW,), jnp.int32),
            pltpu.VMEM((_D,), jnp.float32),
            pltpu.VMEM((_NS * _SPAD,), jnp.float32),
            pltpu.VMEM((_S,), jnp.float32),
            pltpu.SemaphoreType.DMA,
            pltpu.SemaphoreType.DMA,
        ],
    )
    return f(x_flat, ids, w_vec)


def _sc_scatter_y(y_flat, ids, sc_partials_flat):
    mesh = plsc.VectorSubcoreMesh(
        core_axis_name="c", subcore_axis_name="s",
        num_cores=_NC, num_subcores=_NS)

    def body(y_hbm, ids_hbm, part_hbm, out_hbm, y_v, ids_v, part_v, acc2, acc_v,
             sem_y, sem_i, sem_p):
        cid = lax.axis_index("c")
        sid = lax.axis_index("s")
        wid = sid * _NC + cid
        base = wid * _YPW
        cp_y = pltpu.make_async_copy(y_hbm.at[pl.ds(base, _YPW)], y_v, sem_y)
        cp_i = pltpu.make_async_copy(
            ids_hbm.at[pl.ds(_NSC + base, _YPW)], ids_v, sem_i)
        cp_p = pltpu.make_async_copy(
            part_hbm.at[pl.ds(wid * _S, _S)], part_v, sem_p)
        cp_y.start()
        cp_i.start()
        cp_p.start()

        zero = jnp.zeros((_L,), jnp.float32)
        lanes = jnp.arange(_L, dtype=jnp.int32)
        lane_base = lanes * _SPAD

        def zero_body(i, carry):
            acc2[pl.ds(i * _L, _L)] = zero
            return carry

        lax.fori_loop(0, (_NS * _SPAD) // _L, zero_body, 0)
        cp_y.wait()
        cp_i.wait()

        def group_body(g, carry):
            yv = y_v[pl.ds(g * _L, _L)]
            idv = ids_v[pl.ds(g * _L, _L)]
            plsc.addupdate_scatter(acc2, [lane_base + idv], yv)
            return carry

        lax.fori_loop(0, _YG, group_body, 0)
        cp_p.wait()

        def fold_body(cg, carry):
            s = part_v[pl.ds(cg * _L, _L)]
            for r in range(_NS):
                s = s + acc2[pl.ds(r * _SPAD + cg * _L, _L)]
            acc_v[pl.ds(cg * _L, _L)] = s
            return carry

        lax.fori_loop(0, _S // _L, fold_body, 0)
        pltpu.sync_copy(acc_v, out_hbm.at[wid])

    f = pl.kernel(
        body,
        out_type=jax.ShapeDtypeStruct((_NW, _S), jnp.float32),
        mesh=mesh,
        compiler_params=pltpu.CompilerParams(needs_layout_passes=False),
        scratch_types=[
            pltpu.VMEM((_YPW,), jnp.float32),
            pltpu.VMEM((_YPW,), jnp.int32),
            pltpu.VMEM((_S,), jnp.float32),
            pltpu.VMEM((_NS * _SPAD,), jnp.float32),
            pltpu.VMEM((_S,), jnp.float32),
            pltpu.SemaphoreType.DMA,
            pltpu.SemaphoreType.DMA,
            pltpu.SemaphoreType.DMA,
        ],
    )
    return f(y_flat, ids, sc_partials_flat)


def _tc_matvec(x_full, w_row):
    def body(x_ref, w_ref, o_ref):
        y = lax.dot_general(
            w_ref[...], x_ref[...],
            dimension_numbers=(((1,), (1,)), ((), ())),
            preferred_element_type=jnp.float32)
        o_ref[...] = y.reshape(1, 1, _RTC)

    blk0 = _NSC // _RTC

    return pl.pallas_call(
        body,
        grid=(_NBLK,),
        in_specs=[
            pl.BlockSpec((_RTC, _D), lambda i: (i + blk0, 0)),
            pl.BlockSpec((1, _D), lambda i: (0, 0)),
        ],
        out_specs=pl.BlockSpec((1, 1, _RTC), lambda i: (i, 0, 0)),
        out_shape=jax.ShapeDtypeStruct((_NBLK, 1, _RTC), jnp.float32),
        compiler_params=pltpu.CompilerParams(
            dimension_semantics=("parallel",)),
    )(x_full, w_row)


def _combine(partials, b2):
    def body(p_ref, b_ref, o_ref):
        o_ref[...] = jnp.sum(p_ref[...], axis=0, keepdims=True) + b_ref[0, 0]

    return pl.pallas_call(
        body,
        out_shape=jax.ShapeDtypeStruct((1, _S), jnp.float32),
    )(partials, b2)


def kernel(x, batch, W, b):
    x_flat = x.reshape(-1)
    w_vec = W.reshape(-1)
    ids = batch.astype(jnp.int32)
    sc_partials = _sc_partials(x_flat, ids, w_vec)
    y_tc = _tc_matvec(x, W.reshape(1, _D)).reshape(-1)
    partials = _sc_scatter_y(y_tc, ids, sc_partials)
    out2 = _combine(partials, b.reshape(1, 1))
    return out2.reshape(_S)
